# Initial kernel scaffold; baseline (speedup 1.0000x reference)
import jax
import jax.numpy as jnp
from jax.experimental import pallas as pl


def kernel(q_sub, q_rel, r_idx, hidden, edges, n_node, gnn_emb_rel, mapping, Ws, Wr, Wqr, bqr, Wa, ba, Wh):
    raise NotImplementedError("write your pallas kernel here")


# speedup vs baseline: 2.1841x; 2.1841x over previous
import functools

import jax
import jax.numpy as jnp
from jax import lax
from jax.experimental import pallas as pl
from jax.experimental.pallas import tpu as pltpu
from jax.experimental.pallas import tpu_sc as plsc

N_NODE = 10000
E = 320000
B = 1024
D = 128
A = 64

NC = 2
NS = 16
NW = NC * NS
L = 16

CHUNK = 64
E_PER_TILE = 10240
E_PAD = E_PER_TILE * NW
NCHUNK = E_PER_TILE // CHUNK
NPAD = 10240
ROWS_PER_TILE = NPAD // NS
GROUPS = CHUNK // L


def _precompute_tc(hidden, gnn_emb_rel, Ws, Wr, Wqr, bqr2):
    n_uniq = gnn_emb_rel.shape[0]

    def body(h_ref, g_ref, ws_ref, wr_ref, wqr_ref, bqr_ref,
             ps_ref, prw_ref, prq_ref):
        dn = (((1,), (1,)), ((), ()))
        f32 = jnp.float32
        ps_ref[...] = lax.dot_general(h_ref[...], ws_ref[...], dn,
                                      preferred_element_type=f32)
        prw_ref[...] = lax.dot_general(g_ref[...], wr_ref[...], dn,
                                       preferred_element_type=f32)
        prq_ref[...] = lax.dot_general(g_ref[...], wqr_ref[...], dn,
                                       preferred_element_type=f32) + bqr_ref[...]

    return pl.pallas_call(
        body,
        out_shape=[
            jax.ShapeDtypeStruct((N_NODE, A), jnp.float32),
            jax.ShapeDtypeStruct((n_uniq, A), jnp.float32),
            jax.ShapeDtypeStruct((n_uniq, A), jnp.float32),
        ],
    )(hidden, gnn_emb_rel, Ws, Wr, Wqr, bqr2)


def _finish_tc(p0, p1, Wh):
    def body(p0_ref, p1_ref, wh_ref, o_ref):
        s = p0_ref[...] + p1_ref[...]
        o_ref[...] = lax.dot_general(s, wh_ref[...], (((1,), (1,)), ((), ())),
                                     preferred_element_type=jnp.float32)

    return pl.pallas_call(
        body,
        out_shape=jax.ShapeDtypeStruct((N_NODE, D), jnp.float32),
    )(p0, p1, Wh)


def _sc_edges(hid, ps, tr_tab, pr_tab, pq_tab, sub_p, rel_p, ridx_p, obj_p, wab):
    mesh = plsc.VectorSubcoreMesh(core_axis_name="c", subcore_axis_name="s")

    @functools.partial(
        pl.kernel,
        out_type=jax.ShapeDtypeStruct((NC, NPAD, D), jnp.float32),
        mesh=mesh,
        compiler_params=pltpu.CompilerParams(needs_layout_passes=False,
                                             use_tc_tiling_on_sc=False),
        scratch_types=[
            pltpu.VMEM_SHARED((NPAD, D), jnp.float32),
            pltpu.VMEM((CHUNK,), jnp.int32),
            pltpu.VMEM((CHUNK,), jnp.int32),
            pltpu.VMEM((CHUNK,), jnp.int32),
            pltpu.VMEM((CHUNK,), jnp.int32),
            pltpu.VMEM((CHUNK, D), jnp.float32),
            pltpu.VMEM((CHUNK, D), jnp.float32),
            pltpu.VMEM((CHUNK, A), jnp.float32),
            pltpu.VMEM((CHUNK, A), jnp.float32),
            pltpu.VMEM((CHUNK, A), jnp.float32),
            pltpu.VMEM((CHUNK, D), jnp.float32),
            pltpu.VMEM((CHUNK,), jnp.float32),
            pltpu.VMEM((80,), jnp.float32),
            pltpu.SemaphoreType.DMA,
            pltpu.SemaphoreType.DMA,
            pltpu.SemaphoreType.DMA,
            pltpu.SemaphoreType.DMA,
            pltpu.SemaphoreType.DMA,
        ],
    )
    def k(hid_h, ps_h, tr_h, pr_h, pq_h, sub_h, rel_h, ridx_h, obj_h, wab_h,
          out_h, acc, subv, relv, ridxv, objv, hsv, hrv, psv, prv, pqv,
          msgv, alphav, wabv, sem0, sem1, sem2, sem3, sem4):
        c = lax.axis_index("c")
        s = lax.axis_index("s")
        w = s * NC + c

        pltpu.sync_copy(wab_h, wabv)

        zero = jnp.zeros((L,), jnp.float32)

        def zrow(i, _):
            for k2 in range(D // L):
                msgv[i, pl.ds(k2 * L, L)] = zero
            return 0

        lax.fori_loop(0, CHUNK, zrow, 0)

        def zacc(i, _):
            pltpu.sync_copy(
                msgv, acc.at[pl.ds(s * ROWS_PER_TILE + i * CHUNK, CHUNK)])
            return 0

        lax.fori_loop(0, ROWS_PER_TILE // CHUNK, zacc, 0)
        plsc.subcore_barrier()

        iota = lax.iota(jnp.int32, L)
        ba_vec = plsc.load_gather(wabv, [jnp.full((L,), A, dtype=jnp.int32)])

        def chunk_body(ci, _):
            base = w * E_PER_TILE + ci * CHUNK
            pltpu.sync_copy(sub_h.at[pl.ds(base, CHUNK)], subv)
            pltpu.sync_copy(rel_h.at[pl.ds(base, CHUNK)], relv)
            pltpu.sync_copy(ridx_h.at[pl.ds(base, CHUNK)], ridxv)
            pltpu.sync_copy(obj_h.at[pl.ds(base, CHUNK)], objv)
            d0 = pltpu.async_copy(hid_h.at[subv], hsv, sem0)
            d1 = pltpu.async_copy(tr_h.at[relv], hrv, sem1)
            d2 = pltpu.async_copy(ps_h.at[subv], psv, sem2)
            d3 = pltpu.async_copy(pr_h.at[relv], prv, sem3)
            d4 = pltpu.async_copy(pq_h.at[ridxv], pqv, sem4)
            d2.wait()
            d3.wait()
            d4.wait()

            def a_body(a, accs):
                col = jnp.full((L,), a, dtype=jnp.int32)
                waa = plsc.load_gather(wabv, [col])
                out = []
                for g in range(GROUPS):
                    lanes = iota + g * L
                    vs = plsc.load_gather(psv, [lanes, col])
                    vr = plsc.load_gather(prv, [lanes, col])
                    vq = plsc.load_gather(pqv, [lanes, col])
                    out.append(accs[g] + jnp.maximum(vs + vr + vq, 0.0) * waa)
                return tuple(out)

            init = tuple(ba_vec for _ in range(GROUPS))
            accs = lax.fori_loop(0, A, a_body, init)
            for g in range(GROUPS):
                al = 1.0 / (1.0 + jnp.exp(-accs[g]))
                alphav[pl.ds(g * L, L)] = al

            d0.wait()
            d1.wait()

            def m_body(e, _):
                ae = plsc.load_gather(alphav, [jnp.full((L,), e, dtype=jnp.int32)])
                for k2 in range(D // L):
                    sl = pl.ds(k2 * L, L)
                    msgv[e, sl] = hsv[e, sl] * hrv[e, sl] * ae
                return 0

            lax.fori_loop(0, CHUNK, m_body, 0)
            pltpu.sync_copy(msgv, acc.at[objv], add=True)
            return 0

        lax.fori_loop(0, NCHUNK, chunk_body, 0)
        plsc.subcore_barrier()

        def cp(i, _):
            r = s * ROWS_PER_TILE + i * CHUNK
            pltpu.sync_copy(acc.at[pl.ds(r, CHUNK)], out_h.at[c, pl.ds(r, CHUNK)])
            return 0

        lax.fori_loop(0, ROWS_PER_TILE // CHUNK, cp, 0)

    return k(hid, ps, tr_tab, pr_tab, pq_tab, sub_p, rel_p, ridx_p, obj_p, wab)


def kernel(q_sub, q_rel, r_idx, hidden, edges, n_node, gnn_emb_rel, mapping,
           Ws, Wr, Wqr, bqr, Wa, ba, Wh):
    f32, i32 = jnp.float32, jnp.int32
    hidden = hidden.astype(f32)
    sub = edges[:, 0].astype(i32)
    rel = edges[:, 1].astype(i32)
    obj = edges[:, 2].astype(i32)
    r_idx = r_idx.astype(i32)

    pad = E_PAD - E
    sub_p = jnp.concatenate([sub, jnp.zeros((pad,), i32)])
    rel_p = jnp.concatenate([rel, jnp.zeros((pad,), i32)])
    ridx_p = jnp.concatenate([r_idx, jnp.zeros((pad,), i32)])
    obj_p = jnp.concatenate([obj, jnp.full((pad,), N_NODE, i32)])

    bqr2 = bqr.reshape(1, A).astype(f32)
    ps, prw, prq = _precompute_tc(hidden, gnn_emb_rel.astype(f32),
                                  Ws.astype(f32), Wr.astype(f32),
                                  Wqr.astype(f32), bqr2)
    mapping = mapping.astype(i32)
    tr_tab = jnp.take(gnn_emb_rel.astype(f32), mapping, axis=0)
    pr_tab = jnp.take(prw, mapping, axis=0)
    pq_tab = jnp.take(prq, jnp.take(mapping, q_rel.astype(i32)), axis=0)

    wab = jnp.concatenate([Wa.reshape(-1).astype(f32), ba.reshape(-1).astype(f32),
                           jnp.zeros((80 - A - 1,), f32)])

    partials = _sc_edges(hidden, ps, tr_tab, pr_tab, pq_tab,
                         sub_p, rel_p, ridx_p, obj_p, wab)
    return _finish_tc(partials[0, :N_NODE], partials[1, :N_NODE], Wh)


# speedup vs baseline: 3.4376x; 1.5739x over previous
import functools

import jax
import jax.numpy as jnp
from jax import lax
from jax.experimental import pallas as pl
from jax.experimental.pallas import tpu as pltpu
from jax.experimental.pallas import tpu_sc as plsc

N_NODE = 10000
E = 320000
B = 1024
D = 128
A = 64

NC = 2
NS = 16
NW = NC * NS
L = 16

CHUNK = 64
E_PER_TILE = 10240
E_PAD = E_PER_TILE * NW
NCHUNK = E_PER_TILE // CHUNK
NPAD = 10240
ROWS_PER_TILE = NPAD // NS
GROUPS = CHUNK // L


def _precompute_tc(hidden, gnn_emb_rel, Ws, Wr, Wqr, bqr2):
    n_uniq = gnn_emb_rel.shape[0]

    def body(h_ref, g_ref, ws_ref, wr_ref, wqr_ref, bqr_ref,
             ps_ref, prw_ref, prq_ref):
        dn = (((1,), (1,)), ((), ()))
        f32 = jnp.float32
        ps_ref[...] = lax.dot_general(h_ref[...], ws_ref[...], dn,
                                      preferred_element_type=f32)
        prw_ref[...] = lax.dot_general(g_ref[...], wr_ref[...], dn,
                                       preferred_element_type=f32)
        prq_ref[...] = lax.dot_general(g_ref[...], wqr_ref[...], dn,
                                       preferred_element_type=f32) + bqr_ref[...]

    return pl.pallas_call(
        body,
        out_shape=[
            jax.ShapeDtypeStruct((N_NODE, A), jnp.float32),
            jax.ShapeDtypeStruct((n_uniq, A), jnp.float32),
            jax.ShapeDtypeStruct((n_uniq, A), jnp.float32),
        ],
    )(hidden, gnn_emb_rel, Ws, Wr, Wqr, bqr2)


def _finish_tc(p0, p1, Wh):
    def body(p0_ref, p1_ref, wh_ref, o_ref):
        s = p0_ref[...] + p1_ref[...]
        o_ref[...] = lax.dot_general(s, wh_ref[...], (((1,), (1,)), ((), ())),
                                     preferred_element_type=jnp.float32)

    return pl.pallas_call(
        body,
        out_shape=jax.ShapeDtypeStruct((N_NODE, D), jnp.float32),
    )(p0, p1, Wh)


def _sc_edges(hid, ps, tr_tab, pr_tab, pq_tab, idx_all, wab):
    mesh = plsc.VectorSubcoreMesh(core_axis_name="c", subcore_axis_name="s")

    @functools.partial(
        pl.kernel,
        out_type=jax.ShapeDtypeStruct((NC, NPAD, D), jnp.float32),
        mesh=mesh,
        compiler_params=pltpu.CompilerParams(needs_layout_passes=False,
                                             use_tc_tiling_on_sc=False),
        scratch_types=[
            pltpu.VMEM_SHARED((NPAD, D), jnp.float32),
            pltpu.VMEM((2, 4, CHUNK), jnp.int32),
            pltpu.VMEM((2, CHUNK, D), jnp.float32),
            pltpu.VMEM((2, CHUNK, D), jnp.float32),
            pltpu.VMEM((CHUNK, A), jnp.float32),
            pltpu.VMEM((CHUNK, A), jnp.float32),
            pltpu.VMEM((CHUNK, A), jnp.float32),
            pltpu.VMEM((CHUNK,), jnp.float32),
            pltpu.VMEM((80,), jnp.float32),
            pltpu.SemaphoreType.DMA,
            pltpu.SemaphoreType.DMA,
            pltpu.SemaphoreType.DMA,
            pltpu.SemaphoreType.DMA,
            pltpu.SemaphoreType.DMA,
            pltpu.SemaphoreType.DMA,
            pltpu.SemaphoreType.DMA,
            pltpu.SemaphoreType.DMA,
            pltpu.SemaphoreType.DMA,
        ],
    )
    def k(hid_h, ps_h, tr_h, pr_h, pq_h, idx_h, wab_h,
          out_h, acc, idxv, hsv, hrv, psv, prv, pqv, alphav, wabv,
          si0, si1, shs0, shs1, shr0, shr1, sps, spr, spq):
        c = lax.axis_index("c")
        s = lax.axis_index("s")
        w = s * NC + c
        s_idx = (si0, si1)
        s_hs = (shs0, shs1)
        s_hr = (shr0, shr1)

        pltpu.sync_copy(wab_h, wabv)

        zero = jnp.zeros((L,), jnp.float32)

        def zrow(i, _):
            for k2 in range(D // L):
                hsv[0, i, pl.ds(k2 * L, L)] = zero
            return 0

        lax.fori_loop(0, CHUNK, zrow, 0)

        def zacc(i, _):
            pltpu.sync_copy(
                hsv.at[0], acc.at[pl.ds(s * ROWS_PER_TILE + i * CHUNK, CHUNK)])
            return 0

        lax.fori_loop(0, ROWS_PER_TILE // CHUNK, zacc, 0)
        plsc.subcore_barrier()

        cbase = w * NCHUNK

        def issue_idx(ci, b):
            pltpu.async_copy(idx_h.at[cbase + ci], idxv.at[b], s_idx[b])

        def wait_idx(b):
            pltpu.make_async_copy(idx_h.at[cbase], idxv.at[b], s_idx[b]).wait()

        def issue_small(b):
            pltpu.async_copy(ps_h.at[idxv.at[b, 0]], psv, sps)
            pltpu.async_copy(pr_h.at[idxv.at[b, 1]], prv, spr)
            pltpu.async_copy(pq_h.at[idxv.at[b, 2]], pqv, spq)

        def wait_small(b):
            pltpu.make_async_copy(ps_h.at[idxv.at[b, 0]], psv, sps).wait()
            pltpu.make_async_copy(pr_h.at[idxv.at[b, 1]], prv, spr).wait()
            pltpu.make_async_copy(pq_h.at[idxv.at[b, 2]], pqv, spq).wait()

        def issue_big(b):
            pltpu.async_copy(hid_h.at[idxv.at[b, 0]], hsv.at[b], s_hs[b])
            pltpu.async_copy(tr_h.at[idxv.at[b, 1]], hrv.at[b], s_hr[b])

        def wait_big(b):
            pltpu.make_async_copy(hid_h.at[idxv.at[b, 0]], hsv.at[b],
                                  s_hs[b]).wait()
            pltpu.make_async_copy(tr_h.at[idxv.at[b, 1]], hrv.at[b],
                                  s_hr[b]).wait()

        iota = lax.iota(jnp.int32, L)
        ba_vec = plsc.load_gather(wabv, [jnp.full((L,), A, dtype=jnp.int32)])

        issue_idx(0, 0)
        wait_idx(0)
        issue_small(0)
        issue_big(0)
        issue_idx(1, 1)

        def half(ci, b):
            wait_small(b)

            def a_body(a, accs):
                col = jnp.full((L,), a, dtype=jnp.int32)
                waa = plsc.load_gather(wabv, [col])
                out = []
                for g in range(GROUPS):
                    lanes = iota + g * L
                    vs = plsc.load_gather(psv, [lanes, col])
                    vr = plsc.load_gather(prv, [lanes, col])
                    vq = plsc.load_gather(pqv, [lanes, col])
                    out.append(accs[g] + jnp.maximum(vs + vr + vq, 0.0) * waa)
                return tuple(out)

            accs = lax.fori_loop(0, A, a_body,
                                 tuple(ba_vec for _ in range(GROUPS)))
            for g in range(GROUPS):
                al = 1.0 / (1.0 + jnp.exp(-accs[g]))
                alphav[pl.ds(g * L, L)] = al

            @pl.when(ci < NCHUNK - 1)
            def _():
                wait_idx(1 - b)
                issue_small(1 - b)

            wait_big(b)

            def m_body(e, _):
                ae = plsc.load_gather(alphav,
                                      [jnp.full((L,), e, dtype=jnp.int32)])
                for k2 in range(D // L):
                    sl = pl.ds(k2 * L, L)
                    hsv[b, e, sl] = hsv[b, e, sl] * hrv[b, e, sl] * ae
                return 0

            lax.fori_loop(0, CHUNK, m_body, 0)
            pltpu.sync_copy(hsv.at[b], acc.at[idxv.at[b, 3]], add=True)

            @pl.when(ci < NCHUNK - 1)
            def _():
                issue_big(1 - b)

            @pl.when(ci < NCHUNK - 2)
            def _():
                issue_idx(ci + 2, b)

        def pair(p, _):
            half(2 * p, 0)
            half(2 * p + 1, 1)
            return 0

        lax.fori_loop(0, NCHUNK // 2, pair, 0)
        plsc.subcore_barrier()

        def cp(i, _):
            r = s * ROWS_PER_TILE + i * CHUNK
            pltpu.sync_copy(acc.at[pl.ds(r, CHUNK)], out_h.at[c, pl.ds(r, CHUNK)])
            return 0

        lax.fori_loop(0, ROWS_PER_TILE // CHUNK, cp, 0)

    return k(hid, ps, tr_tab, pr_tab, pq_tab, idx_all, wab)


def kernel(q_sub, q_rel, r_idx, hidden, edges, n_node, gnn_emb_rel, mapping,
           Ws, Wr, Wqr, bqr, Wa, ba, Wh):
    f32, i32 = jnp.float32, jnp.int32
    hidden = hidden.astype(f32)
    sub = edges[:, 0].astype(i32)
    rel = edges[:, 1].astype(i32)
    obj = edges[:, 2].astype(i32)
    r_idx = r_idx.astype(i32)

    pad = E_PAD - E
    sub_p = jnp.concatenate([sub, jnp.zeros((pad,), i32)])
    rel_p = jnp.concatenate([rel, jnp.zeros((pad,), i32)])
    ridx_p = jnp.concatenate([r_idx, jnp.zeros((pad,), i32)])
    obj_p = jnp.concatenate([obj, jnp.full((pad,), N_NODE, i32)])
    idx_all = jnp.stack([a.reshape(NW * NCHUNK, CHUNK)
                         for a in (sub_p, rel_p, ridx_p, obj_p)], axis=1)

    bqr2 = bqr.reshape(1, A).astype(f32)
    ps, prw, prq = _precompute_tc(hidden, gnn_emb_rel.astype(f32),
                                  Ws.astype(f32), Wr.astype(f32),
                                  Wqr.astype(f32), bqr2)
    mapping = mapping.astype(i32)
    tr_tab = jnp.take(gnn_emb_rel.astype(f32), mapping, axis=0)
    pr_tab = jnp.take(prw, mapping, axis=0)
    pq_tab = jnp.take(prq, jnp.take(mapping, q_rel.astype(i32)), axis=0)

    wab = jnp.concatenate([Wa.reshape(-1).astype(f32), ba.reshape(-1).astype(f32),
                           jnp.zeros((80 - A - 1,), f32)])

    partials = _sc_edges(hidden, ps, tr_tab, pr_tab, pq_tab, idx_all, wab)
    return _finish_tc(partials[0, :N_NODE], partials[1, :N_NODE], Wh)
